# baseline (speedup 1.0000x reference)
import functools

import jax
import jax.numpy as jnp
from jax import lax
from jax.experimental import pallas as pl
from jax.experimental.pallas import tpu as pltpu
from jax.experimental.pallas import tpu_sc as plsc

_LANES = 16


def kernel(inputs, emb):
    B, H = inputs.shape
    V, D = emb.shape
    NC, NS = 2, 16
    NW = NC * NS
    BPW = B // NW

    flat_cm = lax.optimization_barrier(emb.T.reshape(V * D))
    table = flat_cm.reshape(D, V).T

    mesh = plsc.VectorSubcoreMesh(
        core_axis_name="c", subcore_axis_name="s", num_cores=NC, num_subcores=NS
    )

    @functools.partial(
        pl.kernel,
        out_type=jax.ShapeDtypeStruct((B, D), jnp.float32),
        mesh=mesh,
        scratch_types=[
            pltpu.VMEM((BPW, H), jnp.int32),
            pltpu.VMEM((H, BPW), jnp.int32),
            pltpu.VMEM((BPW, D), jnp.float32),
            pltpu.SemaphoreType.DMA,
        ],
        compiler_params=pltpu.CompilerParams(
            use_tc_tiling_on_sc=False, needs_layout_passes=False
        ),
    )
    def body(idx_hbm, emb_hbm, out_hbm, raw_v, idx_v, acc_v, sem):
        wid = lax.axis_index("s") * NC + lax.axis_index("c")
        pltpu.sync_copy(idx_hbm.at[pl.ds(wid * BPW, BPW)], raw_v)

        def zero_row(i, carry):
            z = jnp.zeros((_LANES,), jnp.float32)
            acc_v[i, pl.ds(0, _LANES)] = z
            acc_v[i, pl.ds(_LANES, _LANES)] = z
            return carry

        lax.fori_loop(0, BPW, zero_row, 0)

        lane = lax.iota(jnp.int32, _LANES)

        def column(j, carry):
            col = jnp.full((_LANES,), j, jnp.int32)

            def chunk(c, carry2):
                v = plsc.load_gather(raw_v, [c * _LANES + lane, col])
                idx_v[j, pl.ds(c * _LANES, _LANES)] = v
                return carry2

            lax.fori_loop(0, BPW // _LANES, chunk, 0)
            pltpu.async_copy(emb_hbm.at[idx_v.at[j]], acc_v, sem, add=True)
            return carry

        lax.fori_loop(0, H, column, 0)

        def drain(j, carry):
            pltpu.make_async_copy(emb_hbm.at[idx_v.at[j]], acc_v, sem).wait()
            return carry

        lax.fori_loop(0, H, drain, 0)

        pltpu.sync_copy(acc_v, out_hbm.at[pl.ds(wid * BPW, BPW)])

    return body(inputs, table)


# speedup vs baseline: 16.5994x; 16.5994x over previous
import functools

import jax
import jax.numpy as jnp
from jax import lax
from jax.experimental import pallas as pl
from jax.experimental.pallas import tpu as pltpu
from jax.experimental.pallas import tpu_sc as plsc

_LANES = 16


def _relayout_body(x_ref, o_ref):
    x = x_ref[...]
    D, C = x.shape
    G = 128 // D
    P = C // G
    xT = x.T
    parts = [xT[a * P:(a + 1) * P, :] for a in range(G)]
    o_ref[...] = jnp.concatenate(parts, axis=1)


def _relayout(embT, V, D, C):
    grid = (V + C - 1) // C
    rows = C * D // 128
    return pl.pallas_call(
        _relayout_body,
        grid=(grid,),
        in_specs=[pl.BlockSpec((D, C), lambda i: (0, i))],
        out_specs=pl.BlockSpec((rows, 128), lambda i: (i, 0)),
        out_shape=jax.ShapeDtypeStruct((grid * rows, 128), jnp.float32),
    )(embT)


def kernel(inputs, emb):
    B, H = inputs.shape
    V, D = emb.shape
    NC, NS = 2, 16
    NW = NC * NS
    BPW = B // NW

    C = 4096
    G = 128 // D
    P = C // G
    PSH = P.bit_length() - 1
    t128 = _relayout(emb.T, V, D, C)
    table = t128.reshape(t128.shape[0] * G, D)

    mesh = plsc.VectorSubcoreMesh(
        core_axis_name="c", subcore_axis_name="s", num_cores=NC, num_subcores=NS
    )

    @functools.partial(
        pl.kernel,
        out_type=jax.ShapeDtypeStruct((B, D), jnp.float32),
        mesh=mesh,
        scratch_types=[
            pltpu.VMEM((BPW, H), jnp.int32),
            pltpu.VMEM((H, BPW), jnp.int32),
            pltpu.VMEM((BPW, D), jnp.float32),
            pltpu.SemaphoreType.DMA,
        ],
        compiler_params=pltpu.CompilerParams(
            use_tc_tiling_on_sc=False, needs_layout_passes=False
        ),
    )
    def body(idx_hbm, emb_hbm, out_hbm, raw_v, idx_v, acc_v, sem):
        wid = lax.axis_index("s") * NC + lax.axis_index("c")
        pltpu.sync_copy(idx_hbm.at[pl.ds(wid * BPW, BPW)], raw_v)

        def zero_row(i, carry):
            z = jnp.zeros((_LANES,), jnp.float32)
            acc_v[i, pl.ds(0, _LANES)] = z
            acc_v[i, pl.ds(_LANES, _LANES)] = z
            return carry

        lax.fori_loop(0, BPW, zero_row, 0)

        lane = lax.iota(jnp.int32, _LANES)

        def column(j, carry):
            col = jnp.full((_LANES,), j, jnp.int32)

            def chunk(c, carry2):
                v = plsc.load_gather(raw_v, [c * _LANES + lane, col])
                u = v & (C - 1)
                pv = (v - u) + G * (u & (P - 1)) + lax.shift_right_logical(u, PSH)
                idx_v[j, pl.ds(c * _LANES, _LANES)] = pv
                return carry2

            lax.fori_loop(0, BPW // _LANES, chunk, 0)
            pltpu.async_copy(emb_hbm.at[idx_v.at[j]], acc_v, sem, add=True)
            return carry

        lax.fori_loop(0, H, column, 0)

        def drain(j, carry):
            pltpu.make_async_copy(emb_hbm.at[idx_v.at[j]], acc_v, sem).wait()
            return carry

        lax.fori_loop(0, H, drain, 0)

        pltpu.sync_copy(acc_v, out_hbm.at[pl.ds(wid * BPW, BPW)])

    return body(inputs, table)


# speedup vs baseline: 18.9900x; 1.1440x over previous
import functools

import jax
import jax.numpy as jnp
from jax import lax
from jax.experimental import pallas as pl
from jax.experimental.pallas import tpu as pltpu
from jax.experimental.pallas import tpu_sc as plsc

_LANES = 16


def _relayout_body(x_ref, o_ref):
    x = x_ref[...]
    D, C = x.shape
    G = 128 // D
    P = C // G
    xT = x.T
    parts = [xT[a * P:(a + 1) * P, :] for a in range(G)]
    o_ref[...] = jnp.concatenate(parts, axis=1)


def _relayout(embT, V, D, C):
    grid = (V + C - 1) // C
    rows = C * D // 128
    return pl.pallas_call(
        _relayout_body,
        grid=(grid,),
        in_specs=[pl.BlockSpec((D, C), lambda i: (0, i))],
        out_specs=pl.BlockSpec((rows, 128), lambda i: (i, 0)),
        out_shape=jax.ShapeDtypeStruct((grid * rows, 128), jnp.float32),
    )(embT)


def kernel(inputs, emb):
    B, H = inputs.shape
    V, D = emb.shape
    NC, NS = 2, 16
    NW = NC * NS
    BPW = B // NW

    C = 16384
    G = 128 // D
    P = C // G
    PSH = P.bit_length() - 1
    t128 = _relayout(emb.T, V, D, C)
    table = t128.reshape(t128.shape[0] * G, D)

    mesh = plsc.VectorSubcoreMesh(
        core_axis_name="c", subcore_axis_name="s", num_cores=NC, num_subcores=NS
    )

    @functools.partial(
        pl.kernel,
        out_type=jax.ShapeDtypeStruct((B, D), jnp.float32),
        mesh=mesh,
        scratch_types=[
            pltpu.VMEM((BPW, H), jnp.int32),
            pltpu.VMEM((H, BPW), jnp.int32),
            pltpu.VMEM((BPW, D), jnp.float32),
            pltpu.SemaphoreType.DMA,
        ],
        compiler_params=pltpu.CompilerParams(
            use_tc_tiling_on_sc=False, needs_layout_passes=False
        ),
    )
    def body(idx_hbm, emb_hbm, out_hbm, raw_v, idx_v, acc_v, sem):
        wid = lax.axis_index("s") * NC + lax.axis_index("c")
        pltpu.sync_copy(idx_hbm.at[pl.ds(wid * BPW, BPW)], raw_v)

        def zero_row(i, carry):
            z = jnp.zeros((_LANES,), jnp.float32)
            acc_v[i, pl.ds(0, _LANES)] = z
            acc_v[i, pl.ds(_LANES, _LANES)] = z
            return carry

        lax.fori_loop(0, BPW, zero_row, 0)

        lane = lax.iota(jnp.int32, _LANES)

        def column(j, carry):
            col = jnp.full((_LANES,), j, jnp.int32)

            def chunk(c, carry2):
                v = plsc.load_gather(raw_v, [c * _LANES + lane, col])
                u = v & (C - 1)
                pv = (v - u) + G * (u & (P - 1)) + lax.shift_right_logical(u, PSH)
                idx_v[j, pl.ds(c * _LANES, _LANES)] = pv
                return carry2

            lax.fori_loop(0, BPW // _LANES, chunk, 0)
            pltpu.async_copy(emb_hbm.at[idx_v.at[j]], acc_v, sem, add=True)
            return carry

        lax.fori_loop(0, H, column, 0)

        def drain(j, carry):
            pltpu.make_async_copy(emb_hbm.at[idx_v.at[j]], acc_v, sem).wait()
            return carry

        lax.fori_loop(0, H, drain, 0)

        pltpu.sync_copy(acc_v, out_hbm.at[pl.ds(wid * BPW, BPW)])

    return body(inputs, table)
